# baseline (speedup 1.0000x reference)
import functools

import jax
import jax.numpy as jnp
from jax.experimental import pallas as pl
from jax.experimental.pallas import tpu as pltpu
from jax.experimental.pallas import tpu_sc as plsc

TOP_K = 2
CAP_FACTOR = 1.25
SINKHORN_ITERS = 3
AUX_W = 0.01

NWORKERS = 32
SC_CHUNK = 32


def _router_body(T, E, cap, x_ref, wr_ref, dest_ref, src_ref, wm_ref, aux_ref):
    xf = x_ref[...]
    logits = jnp.dot(xf, wr_ref[...], preferred_element_type=jnp.float32)
    m = jnp.max(logits, axis=-1, keepdims=True)
    ex = jnp.exp(logits - m)
    probs = ex / jnp.sum(ex, axis=-1, keepdims=True)
    for _ in range(SINKHORN_ITERS):
        probs = probs / jnp.sum(probs, axis=-1, keepdims=True)
        probs = probs / jnp.sum(probs, axis=0, keepdims=True)
        probs = probs * (T / E)
    lane = jax.lax.broadcasted_iota(jnp.int32, (T, E), 1)
    m1 = jnp.max(probs, axis=-1, keepdims=True)
    i1 = jnp.min(jnp.where(probs == m1, lane, E), axis=-1, keepdims=True)
    probs2 = jnp.where(lane == i1, -1e30, probs)
    m2 = jnp.max(probs2, axis=-1, keepdims=True)
    i2 = jnp.min(jnp.where(probs2 == m2, lane, E), axis=-1, keepdims=True)
    wsum = m1 + m2
    w1 = m1 / wsum
    w2 = m2 / wsum
    oh1 = (lane == i1).astype(jnp.int32)
    oh2 = (lane == i2).astype(jnp.int32)
    rowhist = oh1 + oh2
    cum = rowhist
    sh = 1
    while sh < T:
        shifted = jnp.concatenate(
            [jnp.zeros((sh, E), jnp.int32), cum[: T - sh, :]], axis=0)
        cum = cum + shifted
        sh *= 2
    excl = cum - rowhist
    pos1 = jnp.sum(excl * oh1, axis=-1, keepdims=True)
    pos2 = jnp.sum(excl * oh2, axis=-1, keepdims=True)
    mk1 = pos1 < cap
    mk2 = pos2 < cap
    slot1 = i1 * cap + pos1
    slot2 = i2 * cap + pos2
    dummy = E * cap
    d1 = jnp.where(mk1, slot1, dummy)
    d2 = jnp.where(mk2, slot2, dummy)
    s1 = jnp.where(mk1, slot1, 0)
    s2 = jnp.where(mk2, slot2, 0)
    wm1 = jnp.where(mk1, w1, 0.0)
    wm2 = jnp.where(mk2, w2, 0.0)
    dest_ref[...] = jnp.concatenate([d1, d2], axis=1)
    src_ref[...] = jnp.concatenate([s1, s2], axis=1)
    wm_ref[...] = jnp.concatenate([wm1, wm2], axis=1)
    counts = jnp.minimum(jnp.sum(rowhist, axis=0, keepdims=True), cap)
    rppe = jnp.mean(probs, axis=0, keepdims=True)
    aux = AUX_W * E * jnp.sum(rppe * (counts.astype(jnp.float32) / T))
    aux_ref[...] = jnp.full((1, 1), aux, jnp.float32)


def _sc_dispatch_body(T, x_hbm, d1_hbm, d2_hbm, buf_hbm,
                      idx1_v, idx2_v, ra, rb, semL, semS):
    wid = jax.lax.axis_index("s") * 2 + jax.lax.axis_index("c")
    tpw = T // NWORKERS
    nch = tpw // SC_CHUNK
    base = wid * tpw
    pltpu.sync_copy(d1_hbm.at[wid], idx1_v)
    pltpu.sync_copy(d2_hbm.at[wid], idx2_v)
    bufs = [ra, rb]
    loads = [None] * nch
    scat = []
    loads[0] = pltpu.async_copy(x_hbm.at[pl.ds(base, SC_CHUNK)], ra, semL)
    for c in range(nch):
        if c + 1 < nch:
            if c >= 1:
                scat[2 * (c - 1)].wait()
                scat[2 * (c - 1) + 1].wait()
            loads[c + 1] = pltpu.async_copy(
                x_hbm.at[pl.ds(base + (c + 1) * SC_CHUNK, SC_CHUNK)],
                bufs[(c + 1) % 2], semL)
        loads[c].wait()
        r = bufs[c % 2]
        scat.append(pltpu.async_copy(r, buf_hbm.at[idx1_v.at[c]], semS))
        scat.append(pltpu.async_copy(r, buf_hbm.at[idx2_v.at[c]], semS))
    for c in (nch - 2, nch - 1):
        scat[2 * c].wait()
        scat[2 * c + 1].wait()


def _sc_gather_body(NE, eo_hbm, src_hbm, g_hbm, idx_v, ra, rb, semG, semS):
    wid = jax.lax.axis_index("s") * 2 + jax.lax.axis_index("c")
    epw = NE // NWORKERS
    nch = epw // SC_CHUNK
    base = wid * epw
    pltpu.sync_copy(src_hbm.at[wid], idx_v)
    bufs = [ra, rb]
    stores = [None] * nch
    for c in range(nch):
        if c >= 2:
            stores[c - 2].wait()
        pltpu.async_copy(eo_hbm.at[idx_v.at[c]], bufs[c % 2], semG).wait()
        stores[c] = pltpu.async_copy(
            bufs[c % 2], g_hbm.at[pl.ds(base + c * SC_CHUNK, SC_CHUNK)], semS)
    stores[nch - 2].wait()
    stores[nch - 1].wait()


def _ffn_body(xin_ref, w1_ref, b1_ref, w2_ref, b2_ref, out_ref):
    n = pl.program_id(1)
    xb = xin_ref[...].astype(jnp.bfloat16)
    h = jnp.dot(xb, w1_ref[0].astype(jnp.bfloat16),
                preferred_element_type=jnp.float32) + b1_ref[0]
    g = 0.5 * h * (1.0 + jax.lax.erf(h * 0.7071067811865476))
    part = jnp.dot(g.astype(jnp.bfloat16), w2_ref[0].astype(jnp.bfloat16),
                   preferred_element_type=jnp.float32)

    @pl.when(n == 0)
    def _():
        out_ref[...] = part + b2_ref[0]

    @pl.when(n > 0)
    def _():
        out_ref[...] += part


def _combine_body(D, g_ref, wm_ref, y_ref):
    w1 = wm_ref[:, 0:1]
    w2 = wm_ref[:, 1:2]
    a = g_ref[:, :D]
    b = g_ref[:, D:]
    y_ref[...] = (jnp.where(w1 > 0, a * w1, 0.0)
                  + jnp.where(w2 > 0, b * w2, 0.0))


def kernel(x, Wr, W1, b1, W2, b2):
    B, S, D = x.shape
    T = B * S
    E = Wr.shape[1]
    H = W1.shape[2]
    cap = max(int(T * CAP_FACTOR / E), TOP_K)
    xf = x.reshape(T, D)

    dest, src, wm, aux = pl.pallas_call(
        functools.partial(_router_body, T, E, cap),
        out_shape=(
            jax.ShapeDtypeStruct((T, 2), jnp.int32),
            jax.ShapeDtypeStruct((T, 2), jnp.int32),
            jax.ShapeDtypeStruct((T, 2), jnp.float32),
            jax.ShapeDtypeStruct((1, 1), jnp.float32),
        ),
    )(xf, Wr)

    mesh = plsc.VectorSubcoreMesh(core_axis_name="c", subcore_axis_name="s")
    buf = pl.kernel(
        functools.partial(_sc_dispatch_body, T),
        out_type=jax.ShapeDtypeStruct((E * cap + 1, D), jnp.float32),
        mesh=mesh,
        scratch_types=[
            pltpu.VMEM((T // NWORKERS // SC_CHUNK, SC_CHUNK), jnp.int32),
            pltpu.VMEM((T // NWORKERS // SC_CHUNK, SC_CHUNK), jnp.int32),
            pltpu.VMEM((SC_CHUNK, D), jnp.float32),
            pltpu.VMEM((SC_CHUNK, D), jnp.float32),
            pltpu.SemaphoreType.DMA,
            pltpu.SemaphoreType.DMA,
        ],
    )(xf, dest[:, 0].reshape(NWORKERS, T // NWORKERS // SC_CHUNK, SC_CHUNK),
      dest[:, 1].reshape(NWORKERS, T // NWORKERS // SC_CHUNK, SC_CHUNK))

    NT = 4
    hb = H // NT
    eout = pl.pallas_call(
        _ffn_body,
        grid=(E, NT),
        in_specs=[
            pl.BlockSpec((cap, D), lambda e, n: (e, 0)),
            pl.BlockSpec((1, D, hb), lambda e, n: (e, 0, n)),
            pl.BlockSpec((1, 1, hb), lambda e, n: (e, 0, n)),
            pl.BlockSpec((1, hb, D), lambda e, n: (e, n, 0)),
            pl.BlockSpec((1, 1, D), lambda e, n: (e, 0, 0)),
        ],
        out_specs=pl.BlockSpec((cap, D), lambda e, n: (e, 0)),
        out_shape=jax.ShapeDtypeStruct((E * cap, D), jnp.float32),
    )(buf, W1, b1.reshape(E, 1, H), W2, b2.reshape(E, 1, D))

    g = pl.kernel(
        functools.partial(_sc_gather_body, T * 2),
        out_type=jax.ShapeDtypeStruct((T * 2, D), jnp.float32),
        mesh=mesh,
        scratch_types=[
            pltpu.VMEM((T * 2 // NWORKERS // SC_CHUNK, SC_CHUNK), jnp.int32),
            pltpu.VMEM((SC_CHUNK, D), jnp.float32),
            pltpu.VMEM((SC_CHUNK, D), jnp.float32),
            pltpu.SemaphoreType.DMA,
            pltpu.SemaphoreType.DMA,
        ],
    )(eout, src.reshape(NWORKERS, T * 2 // NWORKERS // SC_CHUNK, SC_CHUNK))

    tpb = 256
    y = pl.pallas_call(
        functools.partial(_combine_body, D),
        grid=(T // tpb,),
        in_specs=[
            pl.BlockSpec((tpb, 2 * D), lambda t: (t, 0)),
            pl.BlockSpec((tpb, 2), lambda t: (t, 0)),
        ],
        out_specs=pl.BlockSpec((tpb, D), lambda t: (t, 0)),
        out_shape=jax.ShapeDtypeStruct((T, D), jnp.float32),
    )(g.reshape(T, 2 * D), wm)

    return y.reshape(B, S, D), aux[0, 0]


# speedup vs baseline: 1.9432x; 1.9432x over previous
import functools

import jax
import jax.numpy as jnp
from jax.experimental import pallas as pl
from jax.experimental.pallas import tpu as pltpu
from jax.experimental.pallas import tpu_sc as plsc

TOP_K = 2
CAP_FACTOR = 1.25
SINKHORN_ITERS = 3
AUX_W = 0.01

NWORKERS = 32
SC_CHUNK = 32


def _router_body(T, E, cap, x_ref, wr_ref, dest_ref, src_ref, wm_ref, aux_ref):
    xf = x_ref[...]
    logits = jnp.dot(xf, wr_ref[...], preferred_element_type=jnp.float32)
    m = jnp.max(logits, axis=-1, keepdims=True)
    ex = jnp.exp(logits - m)
    probs = ex / jnp.sum(ex, axis=-1, keepdims=True)
    for _ in range(SINKHORN_ITERS):
        probs = probs / jnp.sum(probs, axis=-1, keepdims=True)
        probs = probs / jnp.sum(probs, axis=0, keepdims=True)
        probs = probs * (T / E)
    lane = jax.lax.broadcasted_iota(jnp.int32, (T, E), 1)
    m1 = jnp.max(probs, axis=-1, keepdims=True)
    i1 = jnp.min(jnp.where(probs == m1, lane, E), axis=-1, keepdims=True)
    probs2 = jnp.where(lane == i1, -1e30, probs)
    m2 = jnp.max(probs2, axis=-1, keepdims=True)
    i2 = jnp.min(jnp.where(probs2 == m2, lane, E), axis=-1, keepdims=True)
    wsum = m1 + m2
    w1 = m1 / wsum
    w2 = m2 / wsum
    oh1 = (lane == i1).astype(jnp.int32)
    oh2 = (lane == i2).astype(jnp.int32)
    rowhist = oh1 + oh2
    cum = rowhist
    sh = 1
    while sh < T:
        shifted = jnp.concatenate(
            [jnp.zeros((sh, E), jnp.int32), cum[: T - sh, :]], axis=0)
        cum = cum + shifted
        sh *= 2
    excl = cum - rowhist
    pos1 = jnp.sum(excl * oh1, axis=-1, keepdims=True)
    pos2 = jnp.sum(excl * oh2, axis=-1, keepdims=True)
    mk1 = pos1 < cap
    mk2 = pos2 < cap
    slot1 = i1 * cap + pos1
    slot2 = i2 * cap + pos2
    dummy = E * cap
    d1 = jnp.where(mk1, slot1, dummy)
    d2 = jnp.where(mk2, slot2, dummy)
    s1 = jnp.where(mk1, slot1, 0)
    s2 = jnp.where(mk2, slot2, 0)
    wm1 = jnp.where(mk1, w1, 0.0)
    wm2 = jnp.where(mk2, w2, 0.0)
    dest_ref[...] = jnp.concatenate([d1, d2], axis=1)
    src_ref[...] = jnp.concatenate([s1, s2], axis=1)
    wm_ref[...] = jnp.concatenate([wm1, wm2], axis=1)
    counts = jnp.minimum(jnp.sum(rowhist, axis=0, keepdims=True), cap)
    rppe = jnp.mean(probs, axis=0, keepdims=True)
    aux = AUX_W * E * jnp.sum(rppe * (counts.astype(jnp.float32) / T))
    aux_ref[...] = jnp.full((1, 1), aux, jnp.float32)


def _dispatch_body(tpb, x_ref, d1_ref, d2_ref, buf_ref):
    t = pl.program_id(0)

    def body(j, _):
        tok = t * tpb + j
        row = x_ref[pl.ds(j, 1), :]
        buf_ref[pl.ds(d1_ref[tok], 1), :] = row
        buf_ref[pl.ds(d2_ref[tok], 1), :] = row
        return 0

    jax.lax.fori_loop(0, tpb, body, 0)


def _tc_combine_body(tpb, eo_ref, s1_ref, s2_ref, w1_ref, w2_ref, y_ref):
    t = pl.program_id(0)

    def body(j, _):
        tok = t * tpb + j
        w1 = w1_ref[tok]
        w2 = w2_ref[tok]
        r1 = jnp.where(w1 > 0, eo_ref[pl.ds(s1_ref[tok], 1), :] * w1, 0.0)
        r2 = jnp.where(w2 > 0, eo_ref[pl.ds(s2_ref[tok], 1), :] * w2, 0.0)
        y_ref[pl.ds(j, 1), :] = r1 + r2
        return 0

    jax.lax.fori_loop(0, tpb, body, 0)


def _sc_dispatch_body(T, x_hbm, d1_hbm, d2_hbm, buf_hbm,
                      idx1_v, idx2_v, ra, rb, semL, semS):
    wid = jax.lax.axis_index("s") * 2 + jax.lax.axis_index("c")
    tpw = T // NWORKERS
    nch = tpw // SC_CHUNK
    base = wid * tpw
    pltpu.sync_copy(d1_hbm.at[wid], idx1_v)
    pltpu.sync_copy(d2_hbm.at[wid], idx2_v)
    bufs = [ra, rb]
    loads = [None] * nch
    scat = []
    loads[0] = pltpu.async_copy(x_hbm.at[pl.ds(base, SC_CHUNK)], ra, semL)
    for c in range(nch):
        if c + 1 < nch:
            if c >= 1:
                scat[2 * (c - 1)].wait()
                scat[2 * (c - 1) + 1].wait()
            loads[c + 1] = pltpu.async_copy(
                x_hbm.at[pl.ds(base + (c + 1) * SC_CHUNK, SC_CHUNK)],
                bufs[(c + 1) % 2], semL)
        loads[c].wait()
        r = bufs[c % 2]
        scat.append(pltpu.async_copy(r, buf_hbm.at[idx1_v.at[c]], semS))
        scat.append(pltpu.async_copy(r, buf_hbm.at[idx2_v.at[c]], semS))
    for c in (nch - 2, nch - 1):
        scat[2 * c].wait()
        scat[2 * c + 1].wait()


def _sc_gather_body(NE, eo_hbm, src_hbm, g_hbm, idx_v, ra, rb, semG, semS):
    wid = jax.lax.axis_index("s") * 2 + jax.lax.axis_index("c")
    epw = NE // NWORKERS
    nch = epw // SC_CHUNK
    base = wid * epw
    pltpu.sync_copy(src_hbm.at[wid], idx_v)
    bufs = [ra, rb]
    stores = [None] * nch
    for c in range(nch):
        if c >= 2:
            stores[c - 2].wait()
        pltpu.async_copy(eo_hbm.at[idx_v.at[c]], bufs[c % 2], semG).wait()
        stores[c] = pltpu.async_copy(
            bufs[c % 2], g_hbm.at[pl.ds(base + c * SC_CHUNK, SC_CHUNK)], semS)
    stores[nch - 2].wait()
    stores[nch - 1].wait()


def _ffn_body(xin_ref, w1_ref, b1_ref, w2_ref, b2_ref, out_ref):
    n = pl.program_id(1)
    xb = xin_ref[...].astype(jnp.bfloat16)
    h = jnp.dot(xb, w1_ref[0].astype(jnp.bfloat16),
                preferred_element_type=jnp.float32) + b1_ref[0]
    g = 0.5 * h * (1.0 + jax.lax.erf(h * 0.7071067811865476))
    part = jnp.dot(g.astype(jnp.bfloat16), w2_ref[0].astype(jnp.bfloat16),
                   preferred_element_type=jnp.float32)

    @pl.when(n == 0)
    def _():
        out_ref[...] = part + b2_ref[0]

    @pl.when(n > 0)
    def _():
        out_ref[...] += part


def _combine_body(D, g_ref, wm_ref, y_ref):
    w1 = wm_ref[:, 0:1]
    w2 = wm_ref[:, 1:2]
    a = g_ref[:, :D]
    b = g_ref[:, D:]
    y_ref[...] = (jnp.where(w1 > 0, a * w1, 0.0)
                  + jnp.where(w2 > 0, b * w2, 0.0))


def kernel(x, Wr, W1, b1, W2, b2):
    B, S, D = x.shape
    T = B * S
    E = Wr.shape[1]
    H = W1.shape[2]
    cap = max(int(T * CAP_FACTOR / E), TOP_K)
    xf = x.reshape(T, D)

    dest, src, wm, aux = pl.pallas_call(
        functools.partial(_router_body, T, E, cap),
        out_shape=(
            jax.ShapeDtypeStruct((T, 2), jnp.int32),
            jax.ShapeDtypeStruct((T, 2), jnp.int32),
            jax.ShapeDtypeStruct((T, 2), jnp.float32),
            jax.ShapeDtypeStruct((1, 1), jnp.float32),
        ),
    )(xf, Wr)

    tpb = 128
    smem = pl.BlockSpec(memory_space=pltpu.SMEM)
    buf = pl.pallas_call(
        functools.partial(_dispatch_body, tpb),
        grid=(T // tpb,),
        in_specs=[
            pl.BlockSpec((tpb, D), lambda t: (t, 0)),
            smem,
            smem,
        ],
        out_specs=pl.BlockSpec((E * cap + 1, D), lambda t: (0, 0)),
        out_shape=jax.ShapeDtypeStruct((E * cap + 1, D), jnp.float32),
    )(xf, dest[:, 0], dest[:, 1])

    NT = 4
    hb = H // NT
    eout = pl.pallas_call(
        _ffn_body,
        grid=(E, NT),
        in_specs=[
            pl.BlockSpec((cap, D), lambda e, n: (e, 0)),
            pl.BlockSpec((1, D, hb), lambda e, n: (e, 0, n)),
            pl.BlockSpec((1, 1, hb), lambda e, n: (e, 0, n)),
            pl.BlockSpec((1, hb, D), lambda e, n: (e, n, 0)),
            pl.BlockSpec((1, 1, D), lambda e, n: (e, 0, 0)),
        ],
        out_specs=pl.BlockSpec((cap, D), lambda e, n: (e, 0)),
        out_shape=jax.ShapeDtypeStruct((E * cap, D), jnp.float32),
    )(buf, W1, b1.reshape(E, 1, H), W2, b2.reshape(E, 1, D))

    y = pl.pallas_call(
        functools.partial(_tc_combine_body, tpb),
        grid=(T // tpb,),
        in_specs=[
            pl.BlockSpec((E * cap, D), lambda t: (0, 0)),
            smem,
            smem,
            smem,
            smem,
        ],
        out_specs=pl.BlockSpec((tpb, D), lambda t: (t, 0)),
        out_shape=jax.ShapeDtypeStruct((T, D), jnp.float32),
    )(eout, src[:, 0], src[:, 1], wm[:, 0], wm[:, 1])

    return y.reshape(B, S, D), aux[0, 0]
